# baseline (speedup 1.0000x reference)
import functools

import jax
import jax.numpy as jnp
from jax import lax
from jax.experimental import pallas as pl
from jax.experimental.pallas import tpu as pltpu
from jax.experimental.pallas import tpu_sc as plsc

_B = 4
_C = 128
_NH = 40962
_NL = 10242
_NLP = 10752
_NW = 32
_PT = _NLP // _NW
_CH = 16
_EPS = 1e-5

_mesh = plsc.VectorSubcoreMesh(core_axis_name="c", subcore_axis_name="s")

_D = _B * _C
_CHN = 8
_ROWS = _CHN * 7
_TCH = _NLP // _CHN
_PF, _PG = 84, 0
_RF, _RG = 84, 0
_IDXPAD = 1400


def _emit_pipeline(nch, nblk, chunk0, nbuf, nf, table_hbm, idx_hbm, out_hbm,
                   idx_v, rows, outs, gsem, wsem, pool):
    pltpu.sync_copy(
        idx_hbm.at[pl.ds(chunk0 * _ROWS, nf * _ROWS)], idx_v)

    def g_start(c, j):
        pltpu.async_copy(
            table_hbm.at[idx_v.at[pl.ds(c * _ROWS, _ROWS)]], rows[j], gsem[j])

    def g_wait(c, j):
        pltpu.make_async_copy(
            table_hbm.at[idx_v.at[pl.ds(c * _ROWS, _ROWS)]], rows[j], gsem[j]
        ).wait()

    if pool:
        def wb_each(c, j):
            yield outs[j], out_hbm.at[pl.ds((chunk0 + c) * _CHN, _CHN)]
    else:
        def wb_each(c, j):
            for k in range(7):
                for b in range(_B):
                    yield (
                        rows[j].at[pl.ds(k * _CHN, _CHN), pl.ds(b * _C, _C)],
                        out_hbm.at[pl.ds(b * _NLP + (chunk0 + c) * _CHN, _CHN),
                                   pl.ds(k * _C, _C)],
                    )

    def wb_start(c, j):
        for src, dst in wb_each(c, j):
            pltpu.async_copy(src, dst, wsem[j])

    def wb_wait(c, j):
        for src, dst in wb_each(c, j):
            pltpu.make_async_copy(src, dst, wsem[j]).wait()

    def compute(j):
        if not pool:
            return
        rv, ov = rows[j], outs[j]

        def grp(cg, _):
            sl = pl.ds(cg * 16, 16)
            for n in range(_CHN):
                base = n * 7
                a = rv[base, sl]
                for t in range(1, 7):
                    a = a + rv[base + t, sl]
                ov[n, sl] = a * (1.0 / 7.0)
            return 0

        lax.fori_loop(0, _D // 16, grp, 0)

    g_start(0, 0)
    g_start(1, 1)
    for jj in range(nbuf):
        if not pool and jj >= nbuf - 2:
            wb_wait(jj + 2 - nbuf, (jj + 2) % nbuf)
        g_start(jj + 2, (jj + 2) % nbuf)
        g_wait(jj, jj)
        compute(jj)
        wb_start(jj, jj)

    def blk(b, _):
        for jj in range(nbuf):
            c = b * nbuf + jj
            if pool:
                g_start(c + 2, (jj + 2) % nbuf)
                g_wait(c, jj)
                wb_wait(c - nbuf, jj)
            else:
                wb_wait(c + 2 - nbuf, (jj + 2) % nbuf)
                g_start(c + 2, (jj + 2) % nbuf)
                g_wait(c, jj)
            compute(jj)
            wb_start(c, jj)
        return 0

    lax.fori_loop(1, nblk - 1, blk, 0)
    for jj in range(nbuf):
        c = nch - nbuf + jj
        if pool:
            if jj < nbuf - 2:
                g_start(c + 2, (jj + 2) % nbuf)
            g_wait(c, jj)
            wb_wait(c - nbuf, jj)
        else:
            if jj < nbuf - 2:
                wb_wait(c + 2 - nbuf, (jj + 2) % nbuf)
                g_start(c + 2, (jj + 2) % nbuf)
            g_wait(c, jj)
        compute(jj)
        wb_start(c, jj)
    for jj in range(nbuf):
        wb_wait(nch - nbuf + jj, jj)


def _split_body(nbuf, f, g, table_hbm, idx_hbm, out_hbm, idx_v, rows, outs,
                gsem, wsem, pool):
    core = lax.axis_index("c")
    sub = lax.axis_index("s")

    @pl.when(core == 0)
    def _run():
        _emit_pipeline(f, f // nbuf, sub * f, nbuf, f, table_hbm, idx_hbm,
                       out_hbm, idx_v, rows, outs, gsem, wsem, pool)


@functools.partial(
    pl.kernel,
    mesh=_mesh,
    out_type=jax.ShapeDtypeStruct((_NLP, _D), jnp.float32),
    scratch_types=[
        pltpu.VMEM((_PF * _ROWS,), jnp.int32),
    ]
    + [pltpu.VMEM((_ROWS, _D), jnp.float32)] * 3
    + [pltpu.VMEM((_CHN, _D), jnp.float32)] * 3
    + [pltpu.SemaphoreType.DMA] * 6,
)
def _pool_gather(table_hbm, idx_hbm, out_hbm, idx_v, r0, r1, r2,
                 o0, o1, o2, g0, g1, g2, w0, w1, w2):
    _split_body(3, _PF, _PG, table_hbm, idx_hbm, out_hbm, idx_v, (r0, r1, r2),
                (o0, o1, o2), (g0, g1, g2), (w0, w1, w2), True)


@functools.partial(
    pl.kernel,
    mesh=_mesh,
    out_type=jax.ShapeDtypeStruct((_B * _NLP, 7 * _C), jnp.float32),
    scratch_types=[
        pltpu.VMEM((_RF * _ROWS,), jnp.int32),
    ]
    + [pltpu.VMEM((_ROWS, _D), jnp.float32)] * 4
    + [pltpu.SemaphoreType.DMA] * 8,
)
def _ring_gather(table_hbm, idx_hbm, out_hbm, idx_v, r0, r1, r2, r3,
                 g0, g1, g2, g3, w0, w1, w2, w3):
    _split_body(4, _RF, _RG, table_hbm, idx_hbm, out_hbm, idx_v,
                (r0, r1, r2, r3), None, (g0, g1, g2, g3), (w0, w1, w2, w3),
                False)


_BLK = 512
_NBLK = (_B * _NLP) // _BLK


def _mm_stats_body(mat_ref, w_ref, b1_ref, out_ref, st_ref):
    j = pl.program_id(0)
    o = (
        jnp.dot(mat_ref[...].astype(jnp.bfloat16), w_ref[...],
                preferred_element_type=jnp.float32)
        + b1_ref[...]
    )
    out_ref[...] = o
    row = j * _BLK + lax.broadcasted_iota(jnp.int32, (_BLK, 1), 0)
    node = row % _NLP
    om = jnp.where(node < _NL, o, 0.0)

    @pl.when(j == 0)
    def _init():
        st_ref[...] = jnp.zeros_like(st_ref)

    st_ref[0:1, :] += jnp.sum(om, axis=0, keepdims=True)
    st_ref[1:2, :] += jnp.sum(om * om, axis=0, keepdims=True)


def _fuse_body(o_ref, x1_ref, sc_ref, sh_ref, wa_ref, wb_ref, bc_ref, y_ref):
    z = o_ref[...] * sc_ref[...] + sh_ref[...]
    z = jnp.where(z >= 0.0, z, 0.2 * z)
    x1t = jnp.transpose(x1_ref[0], (1, 0))
    y = (
        jnp.dot(z.astype(jnp.bfloat16), wa_ref[...],
                preferred_element_type=jnp.float32)
        + jnp.dot(x1t.astype(jnp.bfloat16), wb_ref[...],
                  preferred_element_type=jnp.float32)
        + bc_ref[...]
    )
    y_ref[0] = jnp.transpose(y, (1, 0))


def kernel(x, x1, neigh_orders, pool_neigh_orders, W1, b1, gamma, beta, Wc, bc):
    B, C, Nh = x.shape
    Nl = (Nh + 6) // 4

    xT = jnp.transpose(x, (2, 0, 1)).reshape(Nh, B * C)
    npad = _IDXPAD * _ROWS
    pool_all = jnp.pad(pool_neigh_orders[: Nl * 7], (0, npad - Nl * 7))
    neigh_k = (
        jnp.pad(neigh_orders[: Nl * 7], (0, _NLP * 7 - Nl * 7))
        .reshape(_NLP // _CHN, _CHN, 7)
        .transpose(0, 2, 1)
        .reshape(-1)
    )
    neigh_all = jnp.pad(neigh_k, (0, npad - _NLP * 7))

    xp = _pool_gather(xT, pool_all)
    matg = _ring_gather(xp, neigh_all)

    outT, stats = pl.pallas_call(
        _mm_stats_body,
        grid=(_NBLK,),
        in_specs=[
            pl.BlockSpec((_BLK, 7 * C), lambda j: (j, 0)),
            pl.BlockSpec((7 * C, C), lambda j: (0, 0)),
            pl.BlockSpec((1, C), lambda j: (0, 0)),
        ],
        out_specs=[
            pl.BlockSpec((_BLK, C), lambda j: (j, 0)),
            pl.BlockSpec((8, C), lambda j: (0, 0)),
        ],
        out_shape=[
            jax.ShapeDtypeStruct((_B * _NLP, C), jnp.float32),
            jax.ShapeDtypeStruct((8, C), jnp.float32),
        ],
    )(matg, W1.T.astype(jnp.bfloat16), b1[None, :])

    cnt = jnp.float32(B * Nl)
    mean = stats[0] / cnt
    var = stats[1] / cnt - mean * mean
    scale = gamma * lax.rsqrt(var + _EPS)
    shift = beta - mean * scale

    nj = _NLP // _BLK
    y = pl.pallas_call(
        _fuse_body,
        grid=(B, nj),
        in_specs=[
            pl.BlockSpec((_BLK, C), lambda b, j: (b * nj + j, 0)),
            pl.BlockSpec((1, C, _BLK), lambda b, j: (b, 0, j)),
            pl.BlockSpec((1, C), lambda b, j: (0, 0)),
            pl.BlockSpec((1, C), lambda b, j: (0, 0)),
            pl.BlockSpec((C, C), lambda b, j: (0, 0)),
            pl.BlockSpec((C, C), lambda b, j: (0, 0)),
            pl.BlockSpec((1, C), lambda b, j: (0, 0)),
        ],
        out_specs=pl.BlockSpec((1, C, _BLK), lambda b, j: (b, 0, j)),
        out_shape=jax.ShapeDtypeStruct((B, C, Nl), jnp.float32),
    )(
        outT,
        x1,
        scale[None, :],
        shift[None, :],
        Wc[:, :C].T.astype(jnp.bfloat16),
        Wc[:, C:].T.astype(jnp.bfloat16),
        bc[None, :],
    )
    return y


# speedup vs baseline: 1.0918x; 1.0918x over previous
import functools

import jax
import jax.numpy as jnp
from jax import lax
from jax.experimental import pallas as pl
from jax.experimental.pallas import tpu as pltpu
from jax.experimental.pallas import tpu_sc as plsc

_B = 4
_C = 128
_NH = 40962
_NL = 10242
_NLP = 10752
_NW = 32
_PT = _NLP // _NW
_CH = 16
_EPS = 1e-5

_mesh = plsc.VectorSubcoreMesh(core_axis_name="c", subcore_axis_name="s")

_D = _B * _C
_CHN = 8
_ROWS = _CHN * 7
_TCH = _NLP // _CHN
_PF, _PG = 69, 15
_RF, _RG = 68, 16
_IDXPAD = 1400


def _emit_pipeline(nch, nblk, chunk0, nbuf, nf, table_hbm, idx_hbm, out_hbm,
                   idx_v, rows, outs, gsem, wsem, pool):
    pltpu.sync_copy(
        idx_hbm.at[pl.ds(chunk0 * _ROWS, nf * _ROWS)], idx_v)

    def g_start(c, j):
        pltpu.async_copy(
            table_hbm.at[idx_v.at[pl.ds(c * _ROWS, _ROWS)]], rows[j], gsem[j])

    def g_wait(c, j):
        pltpu.make_async_copy(
            table_hbm.at[idx_v.at[pl.ds(c * _ROWS, _ROWS)]], rows[j], gsem[j]
        ).wait()

    if pool:
        def wb_each(c, j):
            yield outs[j], out_hbm.at[pl.ds((chunk0 + c) * _CHN, _CHN)]
    else:
        def wb_each(c, j):
            for k in range(7):
                for b in range(_B):
                    yield (
                        rows[j].at[pl.ds(k * _CHN, _CHN), pl.ds(b * _C, _C)],
                        out_hbm.at[pl.ds(b * _NLP + (chunk0 + c) * _CHN, _CHN),
                                   pl.ds(k * _C, _C)],
                    )

    def wb_start(c, j):
        for src, dst in wb_each(c, j):
            pltpu.async_copy(src, dst, wsem[j])

    def wb_wait(c, j):
        for src, dst in wb_each(c, j):
            pltpu.make_async_copy(src, dst, wsem[j]).wait()

    def compute(j):
        if not pool:
            return
        rv, ov = rows[j], outs[j]

        def grp(cg, _):
            sl = pl.ds(cg * 16, 16)
            for n in range(_CHN):
                base = n * 7
                a = rv[base, sl]
                for t in range(1, 7):
                    a = a + rv[base + t, sl]
                ov[n, sl] = a * (1.0 / 7.0)
            return 0

        lax.fori_loop(0, _D // 16, grp, 0)

    g_start(0, 0)
    g_start(1, 1)
    for jj in range(nbuf):
        if not pool and jj >= nbuf - 2:
            wb_wait(jj + 2 - nbuf, (jj + 2) % nbuf)
        g_start(jj + 2, (jj + 2) % nbuf)
        g_wait(jj, jj)
        compute(jj)
        wb_start(jj, jj)

    def blk(b, _):
        for jj in range(nbuf):
            c = b * nbuf + jj
            if pool:
                g_start(c + 2, (jj + 2) % nbuf)
                g_wait(c, jj)
                wb_wait(c - nbuf, jj)
            else:
                wb_wait(c + 2 - nbuf, (jj + 2) % nbuf)
                g_start(c + 2, (jj + 2) % nbuf)
                g_wait(c, jj)
            compute(jj)
            wb_start(c, jj)
        return 0

    lax.fori_loop(1, nblk - 1, blk, 0)
    for jj in range(nbuf):
        c = nch - nbuf + jj
        if pool:
            if jj < nbuf - 2:
                g_start(c + 2, (jj + 2) % nbuf)
            g_wait(c, jj)
            wb_wait(c - nbuf, jj)
        else:
            if jj < nbuf - 2:
                wb_wait(c + 2 - nbuf, (jj + 2) % nbuf)
                g_start(c + 2, (jj + 2) % nbuf)
            g_wait(c, jj)
        compute(jj)
        wb_start(c, jj)
    for jj in range(nbuf):
        wb_wait(nch - nbuf + jj, jj)


def _split_body(nbuf, f, g, table_hbm, idx_hbm, out_hbm, idx_v, rows, outs,
                gsem, wsem, pool):
    core = lax.axis_index("c")
    sub = lax.axis_index("s")
    fast = core == 0
    nch = jnp.where(fast, f, g)
    nblk = jnp.where(fast, f // nbuf, g // nbuf)
    chunk0 = jnp.where(fast, sub * f, 16 * f + sub * g)
    _emit_pipeline(nch, nblk, chunk0, nbuf, f, table_hbm, idx_hbm, out_hbm,
                   idx_v, rows, outs, gsem, wsem, pool)


@functools.partial(
    pl.kernel,
    mesh=_mesh,
    out_type=jax.ShapeDtypeStruct((_NLP, _D), jnp.float32),
    scratch_types=[
        pltpu.VMEM((_PF * _ROWS,), jnp.int32),
    ]
    + [pltpu.VMEM((_ROWS, _D), jnp.float32)] * 3
    + [pltpu.VMEM((_CHN, _D), jnp.float32)] * 3
    + [pltpu.SemaphoreType.DMA] * 6,
)
def _pool_gather(table_hbm, idx_hbm, out_hbm, idx_v, r0, r1, r2,
                 o0, o1, o2, g0, g1, g2, w0, w1, w2):
    _split_body(3, _PF, _PG, table_hbm, idx_hbm, out_hbm, idx_v, (r0, r1, r2),
                (o0, o1, o2), (g0, g1, g2), (w0, w1, w2), True)


@functools.partial(
    pl.kernel,
    mesh=_mesh,
    out_type=jax.ShapeDtypeStruct((_B * _NLP, 7 * _C), jnp.float32),
    scratch_types=[
        pltpu.VMEM((_RF * _ROWS,), jnp.int32),
    ]
    + [pltpu.VMEM((_ROWS, _D), jnp.float32)] * 4
    + [pltpu.SemaphoreType.DMA] * 8,
)
def _ring_gather(table_hbm, idx_hbm, out_hbm, idx_v, r0, r1, r2, r3,
                 g0, g1, g2, g3, w0, w1, w2, w3):
    _split_body(4, _RF, _RG, table_hbm, idx_hbm, out_hbm, idx_v,
                (r0, r1, r2, r3), None, (g0, g1, g2, g3), (w0, w1, w2, w3),
                False)


_BLK = 512
_NBLK = (_B * _NLP) // _BLK


def _mm_stats_body(mat_ref, w_ref, b1_ref, out_ref, st_ref):
    j = pl.program_id(0)
    o = (
        jnp.dot(mat_ref[...].astype(jnp.bfloat16), w_ref[...],
                preferred_element_type=jnp.float32)
        + b1_ref[...]
    )
    out_ref[...] = o
    row = j * _BLK + lax.broadcasted_iota(jnp.int32, (_BLK, 1), 0)
    node = row % _NLP
    om = jnp.where(node < _NL, o, 0.0)

    @pl.when(j == 0)
    def _init():
        st_ref[...] = jnp.zeros_like(st_ref)

    st_ref[0:1, :] += jnp.sum(om, axis=0, keepdims=True)
    st_ref[1:2, :] += jnp.sum(om * om, axis=0, keepdims=True)


def _fuse_body(o_ref, x1_ref, sc_ref, sh_ref, wa_ref, wb_ref, bc_ref, y_ref):
    z = o_ref[...] * sc_ref[...] + sh_ref[...]
    z = jnp.where(z >= 0.0, z, 0.2 * z)
    x1t = jnp.transpose(x1_ref[0], (1, 0))
    y = (
        jnp.dot(z.astype(jnp.bfloat16), wa_ref[...],
                preferred_element_type=jnp.float32)
        + jnp.dot(x1t.astype(jnp.bfloat16), wb_ref[...],
                  preferred_element_type=jnp.float32)
        + bc_ref[...]
    )
    y_ref[0] = jnp.transpose(y, (1, 0))


def kernel(x, x1, neigh_orders, pool_neigh_orders, W1, b1, gamma, beta, Wc, bc):
    B, C, Nh = x.shape
    Nl = (Nh + 6) // 4

    xT = jnp.transpose(x, (2, 0, 1)).reshape(Nh, B * C)
    npad = _IDXPAD * _ROWS
    pool_all = jnp.pad(pool_neigh_orders[: Nl * 7], (0, npad - Nl * 7))
    neigh_k = (
        jnp.pad(neigh_orders[: Nl * 7], (0, _NLP * 7 - Nl * 7))
        .reshape(_NLP // _CHN, _CHN, 7)
        .transpose(0, 2, 1)
        .reshape(-1)
    )
    neigh_all = jnp.pad(neigh_k, (0, npad - _NLP * 7))

    xp = _pool_gather(xT, pool_all)
    matg = _ring_gather(xp, neigh_all)

    outT, stats = pl.pallas_call(
        _mm_stats_body,
        grid=(_NBLK,),
        in_specs=[
            pl.BlockSpec((_BLK, 7 * C), lambda j: (j, 0)),
            pl.BlockSpec((7 * C, C), lambda j: (0, 0)),
            pl.BlockSpec((1, C), lambda j: (0, 0)),
        ],
        out_specs=[
            pl.BlockSpec((_BLK, C), lambda j: (j, 0)),
            pl.BlockSpec((8, C), lambda j: (0, 0)),
        ],
        out_shape=[
            jax.ShapeDtypeStruct((_B * _NLP, C), jnp.float32),
            jax.ShapeDtypeStruct((8, C), jnp.float32),
        ],
    )(matg, W1.T.astype(jnp.bfloat16), b1[None, :])

    cnt = jnp.float32(B * Nl)
    mean = stats[0] / cnt
    var = stats[1] / cnt - mean * mean
    scale = gamma * lax.rsqrt(var + _EPS)
    shift = beta - mean * scale

    nj = _NLP // _BLK
    y = pl.pallas_call(
        _fuse_body,
        grid=(B, nj),
        in_specs=[
            pl.BlockSpec((_BLK, C), lambda b, j: (b * nj + j, 0)),
            pl.BlockSpec((1, C, _BLK), lambda b, j: (b, 0, j)),
            pl.BlockSpec((1, C), lambda b, j: (0, 0)),
            pl.BlockSpec((1, C), lambda b, j: (0, 0)),
            pl.BlockSpec((C, C), lambda b, j: (0, 0)),
            pl.BlockSpec((C, C), lambda b, j: (0, 0)),
            pl.BlockSpec((1, C), lambda b, j: (0, 0)),
        ],
        out_specs=pl.BlockSpec((1, C, _BLK), lambda b, j: (b, 0, j)),
        out_shape=jax.ShapeDtypeStruct((B, C, Nl), jnp.float32),
    )(
        outT,
        x1,
        scale[None, :],
        shift[None, :],
        Wc[:, :C].T.astype(jnp.bfloat16),
        Wc[:, C:].T.astype(jnp.bfloat16),
        bc[None, :],
    )
    return y
